# Initial kernel scaffold; baseline (speedup 1.0000x reference)
import jax
import jax.numpy as jnp
from jax.experimental import pallas as pl


def kernel(x, edge_index, W1, b1, W2, b2, rel_emb):
    raise NotImplementedError("write your pallas kernel here")


# speedup vs baseline: 10.9545x; 10.9545x over previous
import functools

import jax
import jax.numpy as jnp
from jax import lax
from jax.experimental import pallas as pl
from jax.experimental.pallas import tpu as pltpu
from jax.experimental.pallas import tpu_sc as plsc

NN = 10000
NE = 320000
RP = 10240
NC = 2
NS = 16
NW = NC * NS
EPW = NE // NW
CH = 80
NCHUNK = EPW // CH
DEGW = 16

_mesh = plsc.VectorSubcoreMesh(core_axis_name="c", subcore_axis_name="s")


def _worker_id():
    return lax.axis_index("c") * NS + lax.axis_index("s")


@functools.partial(
    pl.kernel,
    out_type=jax.ShapeDtypeStruct((NC, RP, DEGW), jnp.float32),
    mesh=_mesh,
    compiler_params=pltpu.CompilerParams(use_tc_tiling_on_sc=False),
    scratch_types=[
        pltpu.VMEM((NCHUNK, CH), jnp.int32),
        pltpu.VMEM((CH, DEGW), jnp.float32),
        pltpu.VMEM_SHARED((RP, DEGW), jnp.float32),
    ],
)
def _sc_deg(dst3_hbm, zdeg_hbm, out_hbm, idxd_v, ones_v, acc_sh):
    cid = lax.axis_index("c")
    sid = lax.axis_index("s")
    wid = _worker_id()
    pltpu.sync_copy(zdeg_hbm.at[pl.ds(sid * 640, 640)],
                    acc_sh.at[pl.ds(sid * 640, 640)])
    pltpu.sync_copy(dst3_hbm.at[wid], idxd_v)

    def fill(i, c):
        ones_v[i] = jnp.full((DEGW,), 1.0, jnp.float32)
        return c
    lax.fori_loop(0, CH, fill, 0)
    plsc.subcore_barrier()

    def chunk(j, c):
        pltpu.sync_copy(ones_v, acc_sh.at[idxd_v.at[j]], add=True)
        return c
    lax.fori_loop(0, NCHUNK, chunk, 0)
    plsc.subcore_barrier()
    pltpu.sync_copy(acc_sh.at[pl.ds(sid * 640, 640)],
                    out_hbm.at[cid, pl.ds(sid * 640, 640)])


def _make_sc_scatter(D):
    @functools.partial(
        pl.kernel,
        out_type=jax.ShapeDtypeStruct((NC, RP, D), jnp.float32),
        mesh=_mesh,
        compiler_params=pltpu.CompilerParams(use_tc_tiling_on_sc=False),
        scratch_types=[
            pltpu.VMEM((NCHUNK, CH), jnp.int32),
            pltpu.VMEM((NCHUNK, CH), jnp.int32),
            pltpu.VMEM((CH, D), jnp.float32),
            pltpu.VMEM_SHARED((RP, D), jnp.float32),
            pltpu.SemaphoreType.DMA,
        ],
    )
    def _sc_scatter(g_hbm, src3_hbm, dst3_hbm, zz_hbm, out_hbm,
                    idxs_v, idxd_v, rows_v, acc_sh, sem):
        cid = lax.axis_index("c")
        sid = lax.axis_index("s")
        wid = _worker_id()
        pltpu.sync_copy(zz_hbm.at[pl.ds(sid * 640, 640)],
                        acc_sh.at[pl.ds(sid * 640, 640)])
        pltpu.sync_copy(src3_hbm.at[wid], idxs_v)
        pltpu.sync_copy(dst3_hbm.at[wid], idxd_v)
        plsc.subcore_barrier()

        def chunk(j, c):
            pltpu.async_copy(g_hbm.at[idxs_v.at[j]], rows_v, sem).wait()
            pltpu.sync_copy(rows_v, acc_sh.at[idxd_v.at[j]], add=True)
            return c
        lax.fori_loop(0, NCHUNK, chunk, 0)
        plsc.subcore_barrier()
        pltpu.sync_copy(acc_sh.at[pl.ds(sid * 640, 640)],
                        out_hbm.at[cid, pl.ds(sid * 640, 640)])
    return _sc_scatter


_sc_scatter128 = _make_sc_scatter(128)
_sc_scatter64 = _make_sc_scatter(64)


NEP = 327680


@functools.partial(
    pl.kernel,
    out_type=(jax.ShapeDtypeStruct((NEP, 64), jnp.float32),
              jax.ShapeDtypeStruct((NEP, 64), jnp.float32)),
    mesh=_mesh,
    compiler_params=pltpu.CompilerParams(use_tc_tiling_on_sc=False),
    scratch_types=[
        pltpu.VMEM((NCHUNK, CH), jnp.int32),
        pltpu.VMEM((NCHUNK, CH), jnp.int32),
        pltpu.VMEM((CH, 64), jnp.float32),
        pltpu.VMEM((CH, 64), jnp.float32),
        pltpu.SemaphoreType.DMA,
        pltpu.SemaphoreType.DMA,
    ],
)
def _sc_decode(zr_hbm, z_hbm, src3_hbm, dst3_hbm, gs_hbm, gt_hbm,
               idxs_v, idxd_v, rows_a, rows_b, sem_a, sem_b):
    wid = _worker_id()
    pltpu.sync_copy(src3_hbm.at[wid], idxs_v)
    pltpu.sync_copy(dst3_hbm.at[wid], idxd_v)

    def chunk(j, c):
        ca = pltpu.async_copy(zr_hbm.at[idxs_v.at[j]], rows_a, sem_a)
        cb = pltpu.async_copy(z_hbm.at[idxd_v.at[j]], rows_b, sem_b)
        ca.wait()
        cb.wait()
        base = wid * EPW + j * CH
        pltpu.sync_copy(rows_a, gs_hbm.at[pl.ds(base, CH)])
        pltpu.sync_copy(rows_b, gt_hbm.at[pl.ds(base, CH)])
        return c
    lax.fori_loop(0, NCHUNK, chunk, 0)


_BR = 1024


def _tc1_body(x_ref, w_ref, degt_ref, o_ref):
    deg = degt_ref[:, 0:1] + degt_ref[:, 1:2] + 1.0
    dinv = lax.rsqrt(deg)
    h = jnp.dot(x_ref[...], w_ref[...], preferred_element_type=jnp.float32)
    o_ref[...] = h * dinv


def _tc1(xp, W1, degt):
    return pl.pallas_call(
        _tc1_body,
        grid=(RP // _BR,),
        in_specs=[
            pl.BlockSpec((_BR, 128), lambda i: (i, 0)),
            pl.BlockSpec((128, 128), lambda i: (0, 0)),
            pl.BlockSpec((_BR, 2), lambda i: (i, 0)),
        ],
        out_specs=pl.BlockSpec((_BR, 128), lambda i: (i, 0)),
        out_shape=jax.ShapeDtypeStruct((RP, 128), jnp.float32),
    )(xp, W1, degt)


def _tc2_body(p_ref, g_ref, degt_ref, b_ref, w_ref, o_ref):
    deg = degt_ref[:, 0:1] + degt_ref[:, 1:2] + 1.0
    dinv = lax.rsqrt(deg)
    a = p_ref[0] + p_ref[1] + g_ref[...]
    h = jnp.maximum(a * dinv + b_ref[...], 0.0)
    o_ref[...] = jnp.dot(h, w_ref[...],
                         preferred_element_type=jnp.float32) * dinv


def _tc2(p1, g1, degt, b1, W2):
    return pl.pallas_call(
        _tc2_body,
        grid=(RP // _BR,),
        in_specs=[
            pl.BlockSpec((NC, _BR, 128), lambda i: (0, i, 0)),
            pl.BlockSpec((_BR, 128), lambda i: (i, 0)),
            pl.BlockSpec((_BR, 2), lambda i: (i, 0)),
            pl.BlockSpec((1, 128), lambda i: (0, 0)),
            pl.BlockSpec((128, 64), lambda i: (0, 0)),
        ],
        out_specs=pl.BlockSpec((_BR, 64), lambda i: (i, 0)),
        out_shape=jax.ShapeDtypeStruct((RP, 64), jnp.float32),
    )(p1, g1, degt, b1, W2)


def _tc3_body(p_ref, g_ref, degt_ref, b_ref, r_ref, z_ref, zr_ref):
    deg = degt_ref[:, 0:1] + degt_ref[:, 1:2] + 1.0
    dinv = lax.rsqrt(deg)
    a = p_ref[0] + p_ref[1] + g_ref[...]
    z = jnp.maximum(a * dinv + b_ref[...], 0.0)
    z_ref[...] = z
    zr_ref[...] = z * r_ref[...]


def _tc3(p2, g2, degt, b2, rel_emb):
    return pl.pallas_call(
        _tc3_body,
        grid=(RP // _BR,),
        in_specs=[
            pl.BlockSpec((NC, _BR, 64), lambda i: (0, i, 0)),
            pl.BlockSpec((_BR, 64), lambda i: (i, 0)),
            pl.BlockSpec((_BR, 2), lambda i: (i, 0)),
            pl.BlockSpec((1, 64), lambda i: (0, 0)),
            pl.BlockSpec((1, 64), lambda i: (0, 0)),
        ],
        out_specs=[
            pl.BlockSpec((_BR, 64), lambda i: (i, 0)),
            pl.BlockSpec((_BR, 64), lambda i: (i, 0)),
        ],
        out_shape=[
            jax.ShapeDtypeStruct((RP, 64), jnp.float32),
            jax.ShapeDtypeStruct((RP, 64), jnp.float32),
        ],
    )(p2, g2, degt, b2, rel_emb)


_EB = 10240


def _tc4_body(a_ref, b_ref, o_ref):
    o_ref[...] = jnp.sum(a_ref[...] * b_ref[...], axis=1).reshape(8, 1280)


def _tc4(gs, gt):
    return pl.pallas_call(
        _tc4_body,
        grid=(NEP // _EB,),
        in_specs=[
            pl.BlockSpec((_EB, 64), lambda i: (i, 0)),
            pl.BlockSpec((_EB, 64), lambda i: (i, 0)),
        ],
        out_specs=pl.BlockSpec((8, 1280), lambda i: (i, 0)),
        out_shape=jax.ShapeDtypeStruct((NEP // 1280, 1280), jnp.float32),
    )(gs, gt)


def kernel(x, edge_index, W1, b1, W2, b2, rel_emb):
    f32 = jnp.float32
    src3 = edge_index[0].reshape(NW, NCHUNK, CH)
    dst3 = edge_index[1].reshape(NW, NCHUNK, CH)
    xp = jnp.pad(x, ((0, RP - NN), (0, 0)))

    degp = _sc_deg(dst3, jnp.zeros((RP, DEGW), f32))
    degt = jnp.concatenate([degp[0, :, :1], degp[1, :, :1]], axis=1)

    g1 = _tc1(xp, W1, degt)
    p1 = _sc_scatter128(g1, src3, dst3, jnp.zeros((RP, 128), f32))
    g2 = _tc2(p1, g1, degt, b1.reshape(1, 128), W2)
    p2 = _sc_scatter64(g2, src3, dst3, jnp.zeros((RP, 64), f32))
    z, zr = _tc3(p2, g2, degt, b2.reshape(1, 64), rel_emb.reshape(1, 64))
    gs, gt = _sc_decode(zr, z, src3, dst3)
    return _tc4(gs, gt).reshape(NEP)[:NE]


# speedup vs baseline: 15.0376x; 1.3727x over previous
import functools

import jax
import jax.numpy as jnp
from jax import lax
from jax.experimental import pallas as pl
from jax.experimental.pallas import tpu as pltpu
from jax.experimental.pallas import tpu_sc as plsc

NN = 10000
NE = 320000
RP = 10240
NC = 2
NS = 16
NW = NC * NS
EPW = NE // NW
CH = 80
NCHUNK = EPW // CH
NPAIR = NE // 2
NPRP = 163840
DEGW = 16

_mesh = plsc.VectorSubcoreMesh(core_axis_name="c", subcore_axis_name="s")
_sc_params = pltpu.CompilerParams(use_tc_tiling_on_sc=False)


def _worker_id():
    return lax.axis_index("c") * NS + lax.axis_index("s")


@functools.partial(
    pl.kernel,
    out_type=jax.ShapeDtypeStruct((NC, RP, DEGW), jnp.float32),
    mesh=_mesh,
    compiler_params=_sc_params,
    scratch_types=[
        pltpu.VMEM((NCHUNK, CH), jnp.int32),
        pltpu.VMEM((CH, DEGW), jnp.float32),
        pltpu.VMEM_SHARED((RP, DEGW), jnp.float32),
    ],
)
def _sc_deg(dst3_hbm, zdeg_hbm, out_hbm, idxd_v, ones_v, acc_sh):
    cid = lax.axis_index("c")
    sid = lax.axis_index("s")
    wid = _worker_id()
    pltpu.sync_copy(zdeg_hbm.at[pl.ds(sid * 640, 640)],
                    acc_sh.at[pl.ds(sid * 640, 640)])
    pltpu.sync_copy(dst3_hbm.at[wid], idxd_v)

    def fill(i, c):
        ones_v[i] = jnp.full((DEGW,), 1.0, jnp.float32)
        return c
    lax.fori_loop(0, CH, fill, 0)
    plsc.subcore_barrier()

    def chunk(j, c):
        pltpu.sync_copy(ones_v, acc_sh.at[idxd_v.at[j]], add=True)
        return c
    lax.fori_loop(0, NCHUNK, chunk, 0)
    plsc.subcore_barrier()
    pltpu.sync_copy(acc_sh.at[pl.ds(sid * 640, 640)],
                    out_hbm.at[cid, pl.ds(sid * 640, 640)])


def _make_sc_scatter(D):
    @functools.partial(
        pl.kernel,
        out_type=jax.ShapeDtypeStruct((NC, RP, D), jnp.float32),
        mesh=_mesh,
        compiler_params=_sc_params,
        scratch_types=[
            pltpu.VMEM((NCHUNK, CH), jnp.int32),
            pltpu.VMEM((NCHUNK, CH), jnp.int32),
            pltpu.VMEM((CH, D), jnp.float32),
            pltpu.VMEM((CH, D), jnp.float32),
            pltpu.VMEM_SHARED((RP, D), jnp.float32),
            pltpu.SemaphoreType.DMA,
            pltpu.SemaphoreType.DMA,
        ],
    )
    def _sc_scatter(g_hbm, src3_hbm, dst3_hbm, zz_hbm, out_hbm,
                    idxs_v, idxd_v, rows_a, rows_b, acc_sh, sem_a, sem_b):
        cid = lax.axis_index("c")
        sid = lax.axis_index("s")
        wid = _worker_id()
        pltpu.sync_copy(zz_hbm.at[pl.ds(sid * 640, 640)],
                        acc_sh.at[pl.ds(sid * 640, 640)])
        pltpu.sync_copy(src3_hbm.at[wid], idxs_v)
        pltpu.sync_copy(dst3_hbm.at[wid], idxd_v)
        plsc.subcore_barrier()

        pltpu.async_copy(g_hbm.at[idxs_v.at[0]], rows_a, sem_a)

        def pair(k, c):
            j0 = 2 * k
            pltpu.make_async_copy(g_hbm.at[idxs_v.at[j0]], rows_a, sem_a).wait()
            pltpu.async_copy(g_hbm.at[idxs_v.at[j0 + 1]], rows_b, sem_b)
            pltpu.sync_copy(rows_a, acc_sh.at[idxd_v.at[j0]], add=True)
            pltpu.make_async_copy(g_hbm.at[idxs_v.at[j0 + 1]], rows_b,
                                  sem_b).wait()
            pltpu.async_copy(g_hbm.at[idxs_v.at[j0 + 2]], rows_a, sem_a)
            pltpu.sync_copy(rows_b, acc_sh.at[idxd_v.at[j0 + 1]], add=True)
            return c
        lax.fori_loop(0, (NCHUNK - 1) // 2, pair, 0)
        pltpu.make_async_copy(g_hbm.at[idxs_v.at[NCHUNK - 1]], rows_a,
                              sem_a).wait()
        pltpu.sync_copy(rows_a, acc_sh.at[idxd_v.at[NCHUNK - 1]], add=True)

        plsc.subcore_barrier()
        pltpu.sync_copy(acc_sh.at[pl.ds(sid * 640, 640)],
                        out_hbm.at[cid, pl.ds(sid * 640, 640)])
    return _sc_scatter


_sc_scatter128 = _make_sc_scatter(128)
_sc_scatter64 = _make_sc_scatter(64)


PPW = NPAIR // NW
CPR = CH // 2


@functools.partial(
    pl.kernel,
    out_type=(jax.ShapeDtypeStruct((NPRP, 128), jnp.float32),
              jax.ShapeDtypeStruct((NPRP, 128), jnp.float32)),
    mesh=_mesh,
    compiler_params=_sc_params,
    scratch_types=[
        pltpu.VMEM((NCHUNK, CPR), jnp.int32),
        pltpu.VMEM((NCHUNK, CPR), jnp.int32),
        pltpu.VMEM((NCHUNK, CPR), jnp.int32),
        pltpu.VMEM((NCHUNK, CPR), jnp.int32),
        [pltpu.VMEM((CPR, 64), jnp.float32)] * 4,
        [pltpu.VMEM((CPR, 64), jnp.float32)] * 4,
        pltpu.SemaphoreType.DMA,
        pltpu.SemaphoreType.DMA,
        pltpu.SemaphoreType.DMA,
        pltpu.SemaphoreType.DMA,
    ],
)
def _sc_decode(zr_hbm, z_hbm, srca_hbm, srcb_hbm, dsta_hbm, dstb_hbm,
               gs_hbm, gt_hbm,
               idxsa, idxsb, idxda, idxdb, set0, set1,
               gsem0, gsem1, wsem0, wsem1):
    wid = _worker_id()
    pltpu.sync_copy(srca_hbm.at[wid], idxsa)
    pltpu.sync_copy(srcb_hbm.at[wid], idxsb)
    pltpu.sync_copy(dsta_hbm.at[wid], idxda)
    pltpu.sync_copy(dstb_hbm.at[wid], idxdb)
    rbase = wid * PPW

    def gath(j, bufs, gsem):
        sa, sb, ta, tb = bufs
        pltpu.async_copy(zr_hbm.at[idxsa.at[j]], sa, gsem)
        pltpu.async_copy(zr_hbm.at[idxsb.at[j]], sb, gsem)
        pltpu.async_copy(z_hbm.at[idxda.at[j]], ta, gsem)
        pltpu.async_copy(z_hbm.at[idxdb.at[j]], tb, gsem)

    def gwait(j, bufs, gsem):
        sa, sb, ta, tb = bufs
        pltpu.make_async_copy(zr_hbm.at[idxsa.at[j]], sa, gsem).wait()
        pltpu.make_async_copy(zr_hbm.at[idxsb.at[j]], sb, gsem).wait()
        pltpu.make_async_copy(z_hbm.at[idxda.at[j]], ta, gsem).wait()
        pltpu.make_async_copy(z_hbm.at[idxdb.at[j]], tb, gsem).wait()

    def _wdsts(j):
        row = pl.ds(rbase + j * CPR, CPR)
        lo, hi = pl.ds(0, 64), pl.ds(64, 64)
        return (gs_hbm.at[row, lo], gs_hbm.at[row, hi],
                gt_hbm.at[row, lo], gt_hbm.at[row, hi])

    def wissue(j, bufs, wsem):
        for b, d in zip(bufs, _wdsts(j)):
            pltpu.async_copy(b, d, wsem)

    def wwait(j, bufs, wsem):
        for b, d in zip(bufs, _wdsts(j)):
            pltpu.make_async_copy(b, d, wsem).wait()

    gath(0, set0, gsem0)

    def pair(k, c):
        j0 = 2 * k
        j1 = j0 + 1
        gwait(j0, set0, gsem0)

        @pl.when(k > 0)
        def _():
            wwait(j0 - 1, set1, wsem1)
        gath(j1, set1, gsem1)
        wissue(j0, set0, wsem0)
        gwait(j1, set1, gsem1)
        wwait(j0, set0, wsem0)
        gath(j1 + 1, set0, gsem0)
        wissue(j1, set1, wsem1)
        return c
    lax.fori_loop(0, (NCHUNK - 1) // 2, pair, 0)
    gwait(NCHUNK - 1, set0, gsem0)
    wwait(NCHUNK - 2, set1, wsem1)
    wissue(NCHUNK - 1, set0, wsem0)
    wwait(NCHUNK - 1, set0, wsem0)


_BR = 1024


def _dinv_of(degp_ref):
    deg = degp_ref[0][:, 0:1] + degp_ref[1][:, 0:1] + 1.0
    return lax.rsqrt(deg)


def _tc1_body(x_ref, w_ref, degp_ref, o_ref):
    h = jnp.dot(x_ref[...], w_ref[...], preferred_element_type=jnp.float32)
    o_ref[...] = h * _dinv_of(degp_ref)


def _tc1(xp, W1, degp):
    return pl.pallas_call(
        _tc1_body,
        grid=(RP // _BR,),
        in_specs=[
            pl.BlockSpec((_BR, 128), lambda i: (i, 0)),
            pl.BlockSpec((128, 128), lambda i: (0, 0)),
            pl.BlockSpec((NC, _BR, DEGW), lambda i: (0, i, 0)),
        ],
        out_specs=pl.BlockSpec((_BR, 128), lambda i: (i, 0)),
        out_shape=jax.ShapeDtypeStruct((RP, 128), jnp.float32),
    )(xp, W1, degp)


def _tc2_body(p_ref, g_ref, degp_ref, b_ref, w_ref, o_ref):
    dinv = _dinv_of(degp_ref)
    a = p_ref[0] + p_ref[1] + g_ref[...]
    h = jnp.maximum(a * dinv + b_ref[...], 0.0)
    o_ref[...] = jnp.dot(h, w_ref[...],
                         preferred_element_type=jnp.float32) * dinv


def _tc2(p1, g1, degp, b1, W2):
    return pl.pallas_call(
        _tc2_body,
        grid=(RP // _BR,),
        in_specs=[
            pl.BlockSpec((NC, _BR, 128), lambda i: (0, i, 0)),
            pl.BlockSpec((_BR, 128), lambda i: (i, 0)),
            pl.BlockSpec((NC, _BR, DEGW), lambda i: (0, i, 0)),
            pl.BlockSpec((1, 128), lambda i: (0, 0)),
            pl.BlockSpec((128, 64), lambda i: (0, 0)),
        ],
        out_specs=pl.BlockSpec((_BR, 64), lambda i: (i, 0)),
        out_shape=jax.ShapeDtypeStruct((RP, 64), jnp.float32),
    )(p1, g1, degp, b1, W2)


def _tc3_body(p_ref, g_ref, degp_ref, b_ref, r_ref, z_ref, zr_ref):
    dinv = _dinv_of(degp_ref)
    a = p_ref[0] + p_ref[1] + g_ref[...]
    z = jnp.maximum(a * dinv + b_ref[...], 0.0)
    z_ref[...] = z
    zr_ref[...] = z * r_ref[...]


def _tc3(p2, g2, degp, b2, rel_emb):
    return pl.pallas_call(
        _tc3_body,
        grid=(RP // _BR,),
        in_specs=[
            pl.BlockSpec((NC, _BR, 64), lambda i: (0, i, 0)),
            pl.BlockSpec((_BR, 64), lambda i: (i, 0)),
            pl.BlockSpec((NC, _BR, DEGW), lambda i: (0, i, 0)),
            pl.BlockSpec((1, 64), lambda i: (0, 0)),
            pl.BlockSpec((1, 64), lambda i: (0, 0)),
        ],
        out_specs=[
            pl.BlockSpec((_BR, 64), lambda i: (i, 0)),
            pl.BlockSpec((_BR, 64), lambda i: (i, 0)),
        ],
        out_shape=[
            jax.ShapeDtypeStruct((RP, 64), jnp.float32),
            jax.ShapeDtypeStruct((RP, 64), jnp.float32),
        ],
    )(p2, g2, degp, b2, rel_emb)


_PB = 1024


def _tc4_body(a_ref, b_ref, o0_ref, o1_ref):
    s = a_ref[...] * b_ref[...]
    o0_ref[...] = jnp.sum(s[:, :64], axis=1)
    o1_ref[...] = jnp.sum(s[:, 64:], axis=1)


def _tc4(gs, gt):
    return pl.pallas_call(
        _tc4_body,
        grid=(NPRP // _PB,),
        in_specs=[
            pl.BlockSpec((_PB, 128), lambda i: (i, 0)),
            pl.BlockSpec((_PB, 128), lambda i: (i, 0)),
        ],
        out_specs=[
            pl.BlockSpec((_PB,), lambda i: (i,)),
            pl.BlockSpec((_PB,), lambda i: (i,)),
        ],
        out_shape=[
            jax.ShapeDtypeStruct((NPRP,), jnp.float32),
            jax.ShapeDtypeStruct((NPRP,), jnp.float32),
        ],
    )(gs, gt)


def kernel(x, edge_index, W1, b1, W2, b2, rel_emb):
    f32 = jnp.float32
    src = edge_index[0]
    dst = edge_index[1]
    src3 = src.reshape(NW, NCHUNK, CH)
    dst3 = dst.reshape(NW, NCHUNK, CH)
    srca = src[:NPAIR].reshape(NW, NCHUNK, CPR)
    srcb = src[NPAIR:].reshape(NW, NCHUNK, CPR)
    dsta = dst[:NPAIR].reshape(NW, NCHUNK, CPR)
    dstb = dst[NPAIR:].reshape(NW, NCHUNK, CPR)
    xp = jnp.pad(x, ((0, RP - NN), (0, 0)))

    degp = _sc_deg(dst3, jnp.zeros((RP, DEGW), f32))
    g1 = _tc1(xp, W1, degp)
    p1 = _sc_scatter128(g1, src3, dst3, jnp.zeros((RP, 128), f32))
    g2 = _tc2(p1, g1, degp, b1.reshape(1, 128), W2)
    p2 = _sc_scatter64(g2, src3, dst3, jnp.zeros((RP, 64), f32))
    z, zr = _tc3(p2, g2, degp, b2.reshape(1, 64), rel_emb.reshape(1, 64))
    gs, gt = _sc_decode(zr, z, srca, srcb, dsta, dstb)
    o0, o1 = _tc4(gs, gt)
    return jnp.concatenate([o0[:NPAIR], o1[:NPAIR]])


# speedup vs baseline: 21.3150x; 1.4174x over previous
import functools

import jax
import jax.numpy as jnp
from jax import lax
from jax.experimental import pallas as pl
from jax.experimental.pallas import tpu as pltpu
from jax.experimental.pallas import tpu_sc as plsc

NN = 10000
NE = 320000
RP = 10240
NC = 2
NS = 16
NW = NC * NS
EPW = NE // NW
CH = 80
NCHUNK = EPW // CH
DEGW = 16

_mesh = plsc.VectorSubcoreMesh(core_axis_name="c", subcore_axis_name="s")
_sc_params = pltpu.CompilerParams(use_tc_tiling_on_sc=False,
                                  needs_layout_passes=False)


def _worker_id():
    return lax.axis_index("c") * NS + lax.axis_index("s")


@functools.partial(
    pl.kernel,
    out_type=jax.ShapeDtypeStruct((NC, RP, DEGW), jnp.float32),
    mesh=_mesh,
    compiler_params=_sc_params,
    scratch_types=[
        pltpu.VMEM((NCHUNK, CH), jnp.int32),
        pltpu.VMEM((CH, DEGW), jnp.float32),
        pltpu.VMEM_SHARED((RP, DEGW), jnp.float32),
    ],
)
def _sc_deg(dst3_hbm, zdeg_hbm, out_hbm, idxd_v, ones_v, acc_sh):
    cid = lax.axis_index("c")
    sid = lax.axis_index("s")
    wid = _worker_id()
    pltpu.sync_copy(zdeg_hbm.at[pl.ds(sid * 640, 640)],
                    acc_sh.at[pl.ds(sid * 640, 640)])
    pltpu.sync_copy(dst3_hbm.at[wid], idxd_v)

    def fill(i, c):
        ones_v[i] = jnp.full((DEGW,), 1.0, jnp.float32)
        return c
    lax.fori_loop(0, CH, fill, 0)
    plsc.subcore_barrier()

    def chunk(j, c):
        pltpu.sync_copy(ones_v, acc_sh.at[idxd_v.at[j]], add=True)
        return c
    lax.fori_loop(0, NCHUNK, chunk, 0)
    plsc.subcore_barrier()
    pltpu.sync_copy(acc_sh.at[pl.ds(sid * 640, 640)],
                    out_hbm.at[cid, pl.ds(sid * 640, 640)])


def _make_sc_scatter(D):
    @functools.partial(
        pl.kernel,
        out_type=jax.ShapeDtypeStruct((NC, RP, D), jnp.float32),
        mesh=_mesh,
        compiler_params=_sc_params,
        scratch_types=[
            pltpu.VMEM((NCHUNK, CH), jnp.int32),
            pltpu.VMEM((NCHUNK, CH), jnp.int32),
            pltpu.VMEM((CH, D), jnp.float32),
            pltpu.VMEM((CH, D), jnp.float32),
            pltpu.VMEM_SHARED((RP, D), jnp.float32),
            pltpu.SemaphoreType.DMA,
            pltpu.SemaphoreType.DMA,
        ],
    )
    def _sc_scatter(g_hbm, src3_hbm, dst3_hbm, zz_hbm, out_hbm,
                    idxs_v, idxd_v, rows_a, rows_b, acc_sh, sem_a, sem_b):
        cid = lax.axis_index("c")
        sid = lax.axis_index("s")
        wid = _worker_id()
        pltpu.sync_copy(zz_hbm.at[pl.ds(sid * 640, 640)],
                        acc_sh.at[pl.ds(sid * 640, 640)])
        pltpu.sync_copy(src3_hbm.at[wid], idxs_v)
        pltpu.sync_copy(dst3_hbm.at[wid], idxd_v)
        plsc.subcore_barrier()

        pltpu.async_copy(g_hbm.at[idxs_v.at[0]], rows_a, sem_a)

        def pair(k, c):
            j0 = 2 * k
            pltpu.make_async_copy(g_hbm.at[idxs_v.at[j0]], rows_a, sem_a).wait()
            pltpu.async_copy(g_hbm.at[idxs_v.at[j0 + 1]], rows_b, sem_b)
            pltpu.sync_copy(rows_a, acc_sh.at[idxd_v.at[j0]], add=True)
            pltpu.make_async_copy(g_hbm.at[idxs_v.at[j0 + 1]], rows_b,
                                  sem_b).wait()
            pltpu.async_copy(g_hbm.at[idxs_v.at[j0 + 2]], rows_a, sem_a)
            pltpu.sync_copy(rows_b, acc_sh.at[idxd_v.at[j0 + 1]], add=True)
            return c
        lax.fori_loop(0, (NCHUNK - 1) // 2, pair, 0)
        pltpu.make_async_copy(g_hbm.at[idxs_v.at[NCHUNK - 1]], rows_a,
                              sem_a).wait()
        pltpu.sync_copy(rows_a, acc_sh.at[idxd_v.at[NCHUNK - 1]], add=True)

        plsc.subcore_barrier()
        pltpu.sync_copy(acc_sh.at[pl.ds(sid * 640, 640)],
                        out_hbm.at[cid, pl.ds(sid * 640, 640)])
    return _sc_scatter


_sc_scatter128 = _make_sc_scatter(128)
_sc_scatter64 = _make_sc_scatter(64)


@functools.partial(
    pl.kernel,
    out_type=jax.ShapeDtypeStruct((NE,), jnp.float32),
    mesh=_mesh,
    compiler_params=_sc_params,
    scratch_types=[
        pltpu.VMEM((NCHUNK, CH), jnp.int32),
        pltpu.VMEM((NCHUNK, CH), jnp.int32),
        pltpu.VMEM((CH, 64), jnp.float32),
        pltpu.VMEM((CH, 64), jnp.float32),
        pltpu.VMEM((CH, 64), jnp.float32),
        pltpu.VMEM((CH, 64), jnp.float32),
        pltpu.VMEM((CH,), jnp.float32),
        pltpu.VMEM((CH,), jnp.float32),
        pltpu.SemaphoreType.DMA,
        pltpu.SemaphoreType.DMA,
        pltpu.SemaphoreType.DMA,
        pltpu.SemaphoreType.DMA,
    ],
)
def _sc_decode(zr_hbm, z_hbm, src3_hbm, dst3_hbm, out_hbm,
               idxs_v, idxd_v, s0, t0, s1, t1, l0, l1,
               gsem0, gsem1, wsem0, wsem1):
    wid = _worker_id()
    pltpu.sync_copy(src3_hbm.at[wid], idxs_v)
    pltpu.sync_copy(dst3_hbm.at[wid], idxd_v)
    ebase = wid * EPW
    lane = lax.iota(jnp.int32, 16)

    def gath(j, bs, bt, gsem):
        pltpu.async_copy(zr_hbm.at[idxs_v.at[j]], bs, gsem)
        pltpu.async_copy(z_hbm.at[idxd_v.at[j]], bt, gsem)

    def gwait(j, bs, bt, gsem):
        pltpu.make_async_copy(zr_hbm.at[idxs_v.at[j]], bs, gsem).wait()
        pltpu.make_async_copy(z_hbm.at[idxd_v.at[j]], bt, gsem).wait()

    def compute(bs, bt, lbuf):
        def grp(g, c):
            r0 = 16 * g
            res = jnp.zeros((16,), jnp.float32)
            for rr in range(16):
                r = r0 + rr
                v = (bs[r, 0:16] * bt[r, 0:16] + bs[r, 16:32] * bt[r, 16:32]
                     + bs[r, 32:48] * bt[r, 32:48]
                     + bs[r, 48:64] * bt[r, 48:64])
                res = jnp.where(lane == rr, jnp.sum(v), res)
            lbuf[pl.ds(r0, 16)] = res
            return c
        lax.fori_loop(0, CH // 16, grp, 0)

    def wissue(j, lbuf, wsem):
        pltpu.async_copy(lbuf, out_hbm.at[pl.ds(ebase + j * CH, CH)], wsem)

    def wwait(j, lbuf, wsem):
        pltpu.make_async_copy(lbuf, out_hbm.at[pl.ds(ebase + j * CH, CH)],
                              wsem).wait()

    gath(0, s0, t0, gsem0)

    def pair(k, c):
        j0 = 2 * k
        j1 = j0 + 1
        gwait(j0, s0, t0, gsem0)
        gath(j1, s1, t1, gsem1)

        @pl.when(k > 0)
        def _():
            wwait(j0 - 2, l0, wsem0)
        compute(s0, t0, l0)
        wissue(j0, l0, wsem0)
        gwait(j1, s1, t1, gsem1)
        gath(j1 + 1, s0, t0, gsem0)

        @pl.when(k > 0)
        def _():
            wwait(j1 - 2, l1, wsem1)
        compute(s1, t1, l1)
        wissue(j1, l1, wsem1)
        return c
    lax.fori_loop(0, (NCHUNK - 1) // 2, pair, 0)
    gwait(NCHUNK - 1, s0, t0, gsem0)
    wwait(NCHUNK - 3, l0, wsem0)
    compute(s0, t0, l0)
    wissue(NCHUNK - 1, l0, wsem0)
    wwait(NCHUNK - 2, l1, wsem1)
    wwait(NCHUNK - 1, l0, wsem0)


_BR = 1024


def _dinv_of(degp_ref):
    deg = degp_ref[0][:, 0:1] + degp_ref[1][:, 0:1] + 1.0
    return lax.rsqrt(deg)


def _tc1_body(x_ref, w_ref, degp_ref, o_ref):
    h = jnp.dot(x_ref[...], w_ref[...], preferred_element_type=jnp.float32)
    o_ref[...] = h * _dinv_of(degp_ref)


def _tc1(xp, W1, degp):
    return pl.pallas_call(
        _tc1_body,
        grid=(RP // _BR,),
        in_specs=[
            pl.BlockSpec((_BR, 128), lambda i: (i, 0)),
            pl.BlockSpec((128, 128), lambda i: (0, 0)),
            pl.BlockSpec((NC, _BR, DEGW), lambda i: (0, i, 0)),
        ],
        out_specs=pl.BlockSpec((_BR, 128), lambda i: (i, 0)),
        out_shape=jax.ShapeDtypeStruct((RP, 128), jnp.float32),
    )(xp, W1, degp)


def _tc2_body(p_ref, g_ref, degp_ref, b_ref, w_ref, o_ref):
    dinv = _dinv_of(degp_ref)
    a = p_ref[0] + p_ref[1] + g_ref[...]
    h = jnp.maximum(a * dinv + b_ref[...], 0.0)
    o_ref[...] = jnp.dot(h, w_ref[...],
                         preferred_element_type=jnp.float32) * dinv


def _tc2(p1, g1, degp, b1, W2):
    return pl.pallas_call(
        _tc2_body,
        grid=(RP // _BR,),
        in_specs=[
            pl.BlockSpec((NC, _BR, 128), lambda i: (0, i, 0)),
            pl.BlockSpec((_BR, 128), lambda i: (i, 0)),
            pl.BlockSpec((NC, _BR, DEGW), lambda i: (0, i, 0)),
            pl.BlockSpec((1, 128), lambda i: (0, 0)),
            pl.BlockSpec((128, 64), lambda i: (0, 0)),
        ],
        out_specs=pl.BlockSpec((_BR, 64), lambda i: (i, 0)),
        out_shape=jax.ShapeDtypeStruct((RP, 64), jnp.float32),
    )(p1, g1, degp, b1, W2)


def _tc3_body(p_ref, g_ref, degp_ref, b_ref, r_ref, z_ref, zr_ref):
    dinv = _dinv_of(degp_ref)
    a = p_ref[0] + p_ref[1] + g_ref[...]
    z = jnp.maximum(a * dinv + b_ref[...], 0.0)
    z_ref[...] = z
    zr_ref[...] = z * r_ref[...]


def _tc3(p2, g2, degp, b2, rel_emb):
    return pl.pallas_call(
        _tc3_body,
        grid=(RP // _BR,),
        in_specs=[
            pl.BlockSpec((NC, _BR, 64), lambda i: (0, i, 0)),
            pl.BlockSpec((_BR, 64), lambda i: (i, 0)),
            pl.BlockSpec((NC, _BR, DEGW), lambda i: (0, i, 0)),
            pl.BlockSpec((1, 64), lambda i: (0, 0)),
            pl.BlockSpec((1, 64), lambda i: (0, 0)),
        ],
        out_specs=[
            pl.BlockSpec((_BR, 64), lambda i: (i, 0)),
            pl.BlockSpec((_BR, 64), lambda i: (i, 0)),
        ],
        out_shape=[
            jax.ShapeDtypeStruct((RP, 64), jnp.float32),
            jax.ShapeDtypeStruct((RP, 64), jnp.float32),
        ],
    )(p2, g2, degp, b2, rel_emb)


def kernel(x, edge_index, W1, b1, W2, b2, rel_emb):
    f32 = jnp.float32
    src3 = edge_index[0].reshape(NW, NCHUNK, CH)
    dst3 = edge_index[1].reshape(NW, NCHUNK, CH)
    xp = jnp.pad(x, ((0, RP - NN), (0, 0)))

    degp = _sc_deg(dst3, jnp.zeros((RP, DEGW), f32))
    g1 = _tc1(xp, W1, degp)
    p1 = _sc_scatter128(g1, src3, dst3, jnp.zeros((RP, 128), f32))
    g2 = _tc2(p1, g1, degp, b1.reshape(1, 128), W2)
    p2 = _sc_scatter64(g2, src3, dst3, jnp.zeros((RP, 64), f32))
    z, zr = _tc3(p2, g2, degp, b2.reshape(1, 64), rel_emb.reshape(1, 64))
    return _sc_decode(zr, z, src3, dst3)
